# baseline (speedup 1.0000x reference)
import jax
import jax.numpy as jnp
import numpy as np
from jax import lax
from jax.experimental import pallas as pl
from jax.experimental.pallas import tpu as pltpu
from jax.experimental.pallas import tpu_sc as plsc

K = 16
R = 16
D = 128
OUT = 16
NUM_CEL = 2 ** K
B = 16384
B2 = B // 2

NUM_CORES = 2
NUM_SUBCORES = 16

_PACK = np.zeros((R * K, R), dtype=np.float32)
for _r in range(R):
    for _k in range(K):
        _PACK[_r * K + _k, _r] = float(2 ** _k)

_BLK = 2048


def _hash_body(x_ref, w_ref, m_ref, out_ref):
    projT = lax.dot_general(
        w_ref[...], x_ref[...],
        dimension_numbers=(((1,), (1,)), ((), ())),
        preferred_element_type=jnp.float32)
    bits = (projT > 0).astype(jnp.float32)
    hT = lax.dot_general(
        m_ref[...], bits,
        dimension_numbers=(((0,), (0,)), ((), ())),
        preferred_element_type=jnp.float32)
    out_ref[...] = hT.astype(jnp.int32).astype(jnp.int16)


def _compute_hashes(X, Wflat, M):
    return pl.pallas_call(
        _hash_body,
        grid=(B // _BLK,),
        in_specs=[
            pl.BlockSpec((_BLK, D), lambda i: (i, 0)),
            pl.BlockSpec((R * K, D), lambda i: (0, 0)),
            pl.BlockSpec((R * K, R), lambda i: (0, 0)),
        ],
        out_specs=pl.BlockSpec((R, _BLK), lambda i: (0, i)),
        out_shape=jax.ShapeDtypeStruct((R, B), jnp.int16),
    )(X, Wflat, M)


_UNROLL = 8
_NR = R // 2
_SUB = 4096
_SPR = NUM_CEL // _SUB
_NSLOT = 21
_RING = _NSLOT * _SUB


def _sc_body(sketch, hTi, out, ring_v, idx0_v, idx1_v, acc_v, shared,
             r0_sem, r1_sem, i0_sem, i1_sem):
    c = lax.axis_index("c")
    s = lax.axis_index("s")
    j = s // 2
    o = c * (OUT // 2) + j
    rhalf = s % 2
    r0 = rhalf * _NR

    idxb = [idx0_v, idx1_v]
    isem = [i0_sem, i1_sem]
    rsem = [r0_sem, r1_sem]

    def idx_cp(t):
        return pltpu.make_async_copy(hTi.at[r0 + t], idxb[t % 2], isem[t % 2])

    def sub_cp(n):
        t, m = divmod(n, _SPR)
        koff = (n % _NSLOT) * _SUB
        return pltpu.make_async_copy(
            sketch.at[o, r0 + t, pl.ds(m * _SUB, _SUB)],
            ring_v.at[pl.ds(koff, _SUB)],
            rsem[t % 2])

    n_subs = _NR * _SPR
    idx_cp(0).start()
    idx_cp(1).start()
    for n in range(_NSLOT):
        sub_cp(n).start()
    fired = _NSLOT

    for t in range(_NR):
        idx_cp(t).wait()
        for m in range(_SPR):
            sub_cp(t * _SPR + m).wait()
        ib = idxb[t % 2]
        first = (t == 0)
        base_off = ((_SPR * t) % _NSLOT) * _SUB

        @plsc.parallel_loop(0, B2, step=16, unroll=_UNROLL)
        def _gather_loop(i, ib=ib, first=first, base_off=base_off):
            sl = pl.ds(i, 16)
            w = ib[sl]
            idx_e = w & 0xFFFF
            idx_o = lax.shift_right_logical(w, 16)
            pe = base_off + idx_e
            pe = jnp.where(pe >= _RING, pe - _RING, pe)
            po = base_off + idx_o
            po = jnp.where(po >= _RING, po - _RING, po)
            ve = plsc.load_gather(ring_v, [pe])
            vo = plsc.load_gather(ring_v, [po])
            slo = pl.ds(B2 + i, 16)
            if first:
                acc_v[sl] = ve
                acc_v[slo] = vo
            else:
                plsc.addupdate(acc_v.at[sl], ve)
                plsc.addupdate(acc_v.at[slo], vo)

        nxt = min(fired + _SPR, n_subs)
        for n in range(fired, nxt):
            sub_cp(n).start()
        fired = nxt
        if t + 2 < _NR:
            idx_cp(t + 2).start()

    @pl.when(rhalf == 1)
    def _publish():
        pltpu.sync_copy(acc_v, shared.at[j])

    plsc.subcore_barrier()

    @pl.when(rhalf == 0)
    def _combine():
        partner = ring_v.at[pl.ds(0, B)]
        pltpu.sync_copy(shared.at[j], partner)

        def body(i, _):
            base = i * (16 * _UNROLL)
            for u in range(_UNROLL):
                sl = pl.ds(base + u * 16, 16)
                acc_v[sl] = (acc_v[sl] + ring_v[sl]) * (1.0 / R)
            return 0

        lax.fori_loop(0, B // (16 * _UNROLL), body, 0)
        pltpu.sync_copy(acc_v, out.at[o])


def _sc_gather(sketch, hT16):
    hTi = lax.bitcast_convert_type(hT16.reshape(R, B2, 2), jnp.int32)
    mesh = plsc.VectorSubcoreMesh(
        core_axis_name="c", subcore_axis_name="s",
        num_cores=NUM_CORES, num_subcores=NUM_SUBCORES)
    f = pl.kernel(
        _sc_body,
        out_type=jax.ShapeDtypeStruct((OUT, B), jnp.float32),
        mesh=mesh,
        scratch_types=[
            pltpu.VMEM((_RING,), jnp.float32),
            pltpu.VMEM((B2,), jnp.int32),
            pltpu.VMEM((B2,), jnp.int32),
            pltpu.VMEM((B,), jnp.float32),
            pltpu.VMEM_SHARED((OUT // 2, B), jnp.float32),
            pltpu.SemaphoreType.DMA,
            pltpu.SemaphoreType.DMA,
            pltpu.SemaphoreType.DMA,
            pltpu.SemaphoreType.DMA,
        ],
        compiler_params=pltpu.CompilerParams(needs_layout_passes=False),
    )
    return f(sketch, hTi)


def kernel(X, W, sketch):
    Wflat = W.reshape(R * K, D)
    M = jnp.asarray(_PACK)
    hT16 = _compute_hashes(X, Wflat, M)
    out = _sc_gather(sketch, hT16)
    return out.reshape(OUT, 2, B2).transpose(2, 1, 0).reshape(B, OUT)


# speedup vs baseline: 1.0147x; 1.0147x over previous
import jax
import jax.numpy as jnp
import numpy as np
from jax import lax
from jax.experimental import pallas as pl
from jax.experimental.pallas import tpu as pltpu
from jax.experimental.pallas import tpu_sc as plsc

K = 16
R = 16
D = 128
OUT = 16
NUM_CEL = 2 ** K
B = 16384
B2 = B // 2

NUM_CORES = 2
NUM_SUBCORES = 16

_PACK = np.zeros((R * K, R), dtype=np.float32)
for _r in range(R):
    for _k in range(K):
        _PACK[_r * K + _k, _r] = float(2 ** _k)

_BLK = 2048


def _hash_body(x_ref, w_ref, m_ref, out_ref):
    projT = lax.dot_general(
        w_ref[...], x_ref[...],
        dimension_numbers=(((1,), (1,)), ((), ())),
        preferred_element_type=jnp.float32)
    bits = (projT > 0).astype(jnp.float32)
    hT = lax.dot_general(
        m_ref[...], bits,
        dimension_numbers=(((0,), (0,)), ((), ())),
        preferred_element_type=jnp.float32)
    out_ref[...] = hT.astype(jnp.int32).astype(jnp.int16)


def _compute_hashes(X, Wflat, M):
    return pl.pallas_call(
        _hash_body,
        grid=(B // _BLK,),
        in_specs=[
            pl.BlockSpec((_BLK, D), lambda i: (i, 0)),
            pl.BlockSpec((R * K, D), lambda i: (0, 0)),
            pl.BlockSpec((R * K, R), lambda i: (0, 0)),
        ],
        out_specs=pl.BlockSpec((R, _BLK), lambda i: (0, i)),
        out_shape=jax.ShapeDtypeStruct((R, B), jnp.int16),
    )(X, Wflat, M)


_UNROLL = 8
_NR = R // 2
_SUB = 65536
_SPR = NUM_CEL // _SUB
_NSLOT = 1
_RING = _NSLOT * _SUB


def _sc_body(sketch, hTi, out, ring_v, idx0_v, idx1_v, acc_v, shared,
             r0_sem, r1_sem, i0_sem, i1_sem):
    c = lax.axis_index("c")
    s = lax.axis_index("s")
    j = s // 2
    o = c * (OUT // 2) + j
    rhalf = s % 2
    r0 = rhalf * _NR

    idxb = [idx0_v, idx1_v]
    isem = [i0_sem, i1_sem]
    rsem = [r0_sem, r1_sem]

    def idx_cp(t):
        return pltpu.make_async_copy(hTi.at[r0 + t], idxb[t % 2], isem[t % 2])

    def sub_cp(n):
        t, m = divmod(n, _SPR)
        koff = (n % _NSLOT) * _SUB
        return pltpu.make_async_copy(
            sketch.at[o, r0 + t, pl.ds(m * _SUB, _SUB)],
            ring_v.at[pl.ds(koff, _SUB)],
            rsem[t % 2])

    n_subs = _NR * _SPR
    idx_cp(0).start()
    idx_cp(1).start()
    for n in range(_NSLOT):
        sub_cp(n).start()
    fired = _NSLOT

    for t in range(_NR):
        idx_cp(t).wait()
        for m in range(_SPR):
            sub_cp(t * _SPR + m).wait()
        ib = idxb[t % 2]
        first = (t == 0)
        base_off = ((_SPR * t) % _NSLOT) * _SUB

        @plsc.parallel_loop(0, B2, step=16, unroll=_UNROLL)
        def _gather_loop(i, ib=ib, first=first, base_off=base_off):
            sl = pl.ds(i, 16)
            w = ib[sl]
            idx_e = w & 0xFFFF
            idx_o = lax.shift_right_logical(w, 16)
            pe = base_off + idx_e
            pe = jnp.where(pe >= _RING, pe - _RING, pe)
            po = base_off + idx_o
            po = jnp.where(po >= _RING, po - _RING, po)
            ve = plsc.load_gather(ring_v, [pe])
            vo = plsc.load_gather(ring_v, [po])
            slo = pl.ds(B2 + i, 16)
            if first:
                acc_v[sl] = ve
                acc_v[slo] = vo
            else:
                plsc.addupdate(acc_v.at[sl], ve)
                plsc.addupdate(acc_v.at[slo], vo)

        nxt = min(fired + _SPR, n_subs)
        for n in range(fired, nxt):
            sub_cp(n).start()
        fired = nxt
        if t + 2 < _NR:
            idx_cp(t + 2).start()

    @pl.when(rhalf == 1)
    def _publish():
        pltpu.sync_copy(acc_v, shared.at[j])

    plsc.subcore_barrier()

    @pl.when(rhalf == 0)
    def _combine():
        partner = ring_v.at[pl.ds(0, B)]
        pltpu.sync_copy(shared.at[j], partner)

        def body(i, _):
            base = i * (16 * _UNROLL)
            for u in range(_UNROLL):
                sl = pl.ds(base + u * 16, 16)
                acc_v[sl] = (acc_v[sl] + ring_v[sl]) * (1.0 / R)
            return 0

        lax.fori_loop(0, B // (16 * _UNROLL), body, 0)
        pltpu.sync_copy(acc_v, out.at[o])


def _sc_gather(sketch, hT16):
    hTi = lax.bitcast_convert_type(hT16.reshape(R, B2, 2), jnp.int32)
    mesh = plsc.VectorSubcoreMesh(
        core_axis_name="c", subcore_axis_name="s",
        num_cores=NUM_CORES, num_subcores=NUM_SUBCORES)
    f = pl.kernel(
        _sc_body,
        out_type=jax.ShapeDtypeStruct((OUT, B), jnp.float32),
        mesh=mesh,
        scratch_types=[
            pltpu.VMEM((_RING,), jnp.float32),
            pltpu.VMEM((B2,), jnp.int32),
            pltpu.VMEM((B2,), jnp.int32),
            pltpu.VMEM((B,), jnp.float32),
            pltpu.VMEM_SHARED((OUT // 2, B), jnp.float32),
            pltpu.SemaphoreType.DMA,
            pltpu.SemaphoreType.DMA,
            pltpu.SemaphoreType.DMA,
            pltpu.SemaphoreType.DMA,
        ],
        compiler_params=pltpu.CompilerParams(needs_layout_passes=False),
    )
    return f(sketch, hTi)


def kernel(X, W, sketch):
    Wflat = W.reshape(R * K, D)
    M = jnp.asarray(_PACK)
    hT16 = _compute_hashes(X, Wflat, M)
    out = _sc_gather(sketch, hT16)
    return out.reshape(OUT, 2, B2).transpose(2, 1, 0).reshape(B, OUT)


# speedup vs baseline: 1.9458x; 1.9176x over previous
import jax
import jax.numpy as jnp
import numpy as np
from jax import lax
from jax.experimental import pallas as pl
from jax.experimental.pallas import tpu as pltpu
from jax.experimental.pallas import tpu_sc as plsc

K = 16
R = 16
D = 128
OUT = 16
NUM_CEL = 2 ** K
B = 16384
B2 = B // 2

NUM_CORES = 2
NUM_SUBCORES = 16

_PACK = np.zeros((R * K, R), dtype=np.float32)
for _r in range(R):
    for _k in range(K):
        _PACK[_r * K + _k, _r] = float(2 ** _k)

_BLK = 2048


def _hash_block(x_blk, w, m):
    projT = lax.dot_general(
        w, x_blk,
        dimension_numbers=(((1,), (1,)), ((), ())),
        preferred_element_type=jnp.float32)
    bits = (projT > 0).astype(jnp.float32)
    hT = lax.dot_general(
        m, bits,
        dimension_numbers=(((0,), (0,)), ((), ())),
        preferred_element_type=jnp.float32)
    return hT.astype(jnp.int32)


def _hash_body(xlo_ref, xhi_ref, w_ref, m_ref, out_ref):
    lo = _hash_block(xlo_ref[...], w_ref[...], m_ref[...])
    hi = _hash_block(xhi_ref[...], w_ref[...], m_ref[...])
    out_ref[...] = lo | lax.shift_left(hi, 16)


def _compute_hashes(X, Wflat, M):
    nblk = B2 // _BLK
    return pl.pallas_call(
        _hash_body,
        grid=(nblk,),
        in_specs=[
            pl.BlockSpec((_BLK, D), lambda i: (i, 0)),
            pl.BlockSpec((_BLK, D), lambda i, nblk=nblk: (nblk + i, 0)),
            pl.BlockSpec((R * K, D), lambda i: (0, 0)),
            pl.BlockSpec((R * K, R), lambda i: (0, 0)),
        ],
        out_specs=pl.BlockSpec((R, _BLK), lambda i: (0, i)),
        out_shape=jax.ShapeDtypeStruct((R, B2), jnp.int32),
    )(X, X, Wflat, M)


_UNROLL = 8
_NR = R // 2
_SUB = 65536
_SPR = NUM_CEL // _SUB
_NSLOT = 1
_RING = _NSLOT * _SUB


def _sc_body(sketch, hTi, out, ring_v, idx0_v, idx1_v, acc_v, shared,
             r0_sem, r1_sem, i0_sem, i1_sem):
    c = lax.axis_index("c")
    s = lax.axis_index("s")
    j = s // 2
    o = c * (OUT // 2) + j
    rhalf = s % 2
    r0 = rhalf * _NR

    idxb = [idx0_v, idx1_v]
    isem = [i0_sem, i1_sem]
    rsem = [r0_sem, r1_sem]

    def idx_cp(t):
        return pltpu.make_async_copy(hTi.at[r0 + t], idxb[t % 2], isem[t % 2])

    def sub_cp(n):
        t, m = divmod(n, _SPR)
        koff = (n % _NSLOT) * _SUB
        return pltpu.make_async_copy(
            sketch.at[o, r0 + t, pl.ds(m * _SUB, _SUB)],
            ring_v.at[pl.ds(koff, _SUB)],
            rsem[t % 2])

    n_subs = _NR * _SPR
    idx_cp(0).start()
    idx_cp(1).start()
    for n in range(_NSLOT):
        sub_cp(n).start()
    fired = _NSLOT

    for t in range(_NR):
        idx_cp(t).wait()
        for m in range(_SPR):
            sub_cp(t * _SPR + m).wait()
        ib = idxb[t % 2]
        first = (t == 0)
        base_off = ((_SPR * t) % _NSLOT) * _SUB

        @plsc.parallel_loop(0, B2, step=16, unroll=_UNROLL)
        def _gather_loop(i, ib=ib, first=first, base_off=base_off):
            sl = pl.ds(i, 16)
            w = ib[sl]
            idx_e = w & 0xFFFF
            idx_o = lax.shift_right_logical(w, 16)
            pe = base_off + idx_e
            pe = jnp.where(pe >= _RING, pe - _RING, pe)
            po = base_off + idx_o
            po = jnp.where(po >= _RING, po - _RING, po)
            ve = plsc.load_gather(ring_v, [pe])
            vo = plsc.load_gather(ring_v, [po])
            slo = pl.ds(B2 + i, 16)
            if first:
                acc_v[sl] = ve
                acc_v[slo] = vo
            else:
                plsc.addupdate(acc_v.at[sl], ve)
                plsc.addupdate(acc_v.at[slo], vo)

        nxt = min(fired + _SPR, n_subs)
        for n in range(fired, nxt):
            sub_cp(n).start()
        fired = nxt
        if t + 2 < _NR:
            idx_cp(t + 2).start()

    @pl.when(rhalf == 1)
    def _publish():
        pltpu.sync_copy(acc_v, shared.at[j])

    plsc.subcore_barrier()

    @pl.when(rhalf == 0)
    def _combine():
        partner = ring_v.at[pl.ds(0, B)]
        pltpu.sync_copy(shared.at[j], partner)

        def body(i, _):
            base = i * (16 * _UNROLL)
            for u in range(_UNROLL):
                sl = pl.ds(base + u * 16, 16)
                acc_v[sl] = (acc_v[sl] + ring_v[sl]) * (1.0 / R)
            return 0

        lax.fori_loop(0, B // (16 * _UNROLL), body, 0)
        pltpu.sync_copy(acc_v, out.at[o])


def _sc_gather(sketch, hTi):
    mesh = plsc.VectorSubcoreMesh(
        core_axis_name="c", subcore_axis_name="s",
        num_cores=NUM_CORES, num_subcores=NUM_SUBCORES)
    f = pl.kernel(
        _sc_body,
        out_type=jax.ShapeDtypeStruct((OUT, B), jnp.float32),
        mesh=mesh,
        scratch_types=[
            pltpu.VMEM((_RING,), jnp.float32),
            pltpu.VMEM((B2,), jnp.int32),
            pltpu.VMEM((B2,), jnp.int32),
            pltpu.VMEM((B,), jnp.float32),
            pltpu.VMEM_SHARED((OUT // 2, B), jnp.float32),
            pltpu.SemaphoreType.DMA,
            pltpu.SemaphoreType.DMA,
            pltpu.SemaphoreType.DMA,
            pltpu.SemaphoreType.DMA,
        ],
        compiler_params=pltpu.CompilerParams(needs_layout_passes=False),
    )
    return f(sketch, hTi)


def kernel(X, W, sketch):
    Wflat = W.reshape(R * K, D)
    M = jnp.asarray(_PACK)
    hTi = _compute_hashes(X, Wflat, M)
    out = _sc_gather(sketch, hTi)
    return out.T


# speedup vs baseline: 1.9479x; 1.0011x over previous
import jax
import jax.numpy as jnp
import numpy as np
from jax import lax
from jax.experimental import pallas as pl
from jax.experimental.pallas import tpu as pltpu
from jax.experimental.pallas import tpu_sc as plsc

K = 16
R = 16
D = 128
OUT = 16
NUM_CEL = 2 ** K
B = 16384
B2 = B // 2

NUM_CORES = 2
NUM_SUBCORES = 16

_PACK = np.zeros((R * K, R), dtype=np.float32)
for _r in range(R):
    for _k in range(K):
        _PACK[_r * K + _k, _r] = float(2 ** _k)

_BLK = 2048


def _hash_block(x_blk, w, m):
    projT = lax.dot_general(
        w, x_blk,
        dimension_numbers=(((1,), (1,)), ((), ())),
        preferred_element_type=jnp.float32)
    bits = (projT > 0).astype(jnp.float32)
    hT = lax.dot_general(
        m, bits,
        dimension_numbers=(((0,), (0,)), ((), ())),
        preferred_element_type=jnp.float32)
    return hT.astype(jnp.int32)


def _hash_body(xlo_ref, xhi_ref, w_ref, m_ref, out_ref):
    lo = _hash_block(xlo_ref[...], w_ref[...], m_ref[...])
    hi = _hash_block(xhi_ref[...], w_ref[...], m_ref[...])
    out_ref[...] = lo | lax.shift_left(hi, 16)


def _compute_hashes(X, Wflat, M):
    nblk = B2 // _BLK
    return pl.pallas_call(
        _hash_body,
        grid=(nblk,),
        in_specs=[
            pl.BlockSpec((_BLK, D), lambda i: (i, 0)),
            pl.BlockSpec((_BLK, D), lambda i, nblk=nblk: (nblk + i, 0)),
            pl.BlockSpec((R * K, D), lambda i: (0, 0)),
            pl.BlockSpec((R * K, R), lambda i: (0, 0)),
        ],
        out_specs=pl.BlockSpec((R, _BLK), lambda i: (0, i)),
        out_shape=jax.ShapeDtypeStruct((R, B2), jnp.int32),
    )(X, X, Wflat, M)


_UNROLL = 8
_NR = R // 2
_SUB = 32768
_SPR = NUM_CEL // _SUB
_NSLOT = 3
_RING = _NSLOT * _SUB


def _sc_body(sketch, hTi, out, ring_v, idx_v, acc_v, shared,
             r0_sem, r1_sem, i_sem):
    c = lax.axis_index("c")
    s = lax.axis_index("s")
    j = s // 2
    o = c * (OUT // 2) + j
    rhalf = s % 2
    r0 = rhalf * _NR

    rsem = [r0_sem, r1_sem]

    def idx_cp(t):
        return pltpu.make_async_copy(hTi.at[r0 + t], idx_v, i_sem)

    def sub_cp(n):
        t, m = divmod(n, _SPR)
        koff = (n % _NSLOT) * _SUB
        return pltpu.make_async_copy(
            sketch.at[o, r0 + t, pl.ds(m * _SUB, _SUB)],
            ring_v.at[pl.ds(koff, _SUB)],
            rsem[t % 2])

    n_subs = _NR * _SPR
    idx_cp(0).start()
    for n in range(_NSLOT):
        sub_cp(n).start()
    fired = _NSLOT

    for t in range(_NR):
        idx_cp(t).wait()
        for m in range(_SPR):
            sub_cp(t * _SPR + m).wait()
        ib = idx_v
        first = (t == 0)
        base_off = ((_SPR * t) % _NSLOT) * _SUB

        @plsc.parallel_loop(0, B2, step=16, unroll=_UNROLL)
        def _gather_loop(i, ib=ib, first=first, base_off=base_off):
            sl = pl.ds(i, 16)
            w = ib[sl]
            idx_e = w & 0xFFFF
            idx_o = lax.shift_right_logical(w, 16)
            pe = base_off + idx_e
            pe = jnp.where(pe >= _RING, pe - _RING, pe)
            po = base_off + idx_o
            po = jnp.where(po >= _RING, po - _RING, po)
            ve = plsc.load_gather(ring_v, [pe])
            vo = plsc.load_gather(ring_v, [po])
            slo = pl.ds(B2 + i, 16)
            if first:
                acc_v[sl] = ve
                acc_v[slo] = vo
            else:
                plsc.addupdate(acc_v.at[sl], ve)
                plsc.addupdate(acc_v.at[slo], vo)

        nxt = min(fired + _SPR, n_subs)
        for n in range(fired, nxt):
            sub_cp(n).start()
        fired = nxt
        if t + 1 < _NR:
            idx_cp(t + 1).start()

    def _half_combine(acc_base):
        pltpu.sync_copy(shared.at[j], ring_v.at[pl.ds(0, B2)])

        def body(i, _):
            base = i * (16 * _UNROLL)
            for u in range(_UNROLL):
                sl = pl.ds(base + u * 16, 16)
                sla = pl.ds(acc_base + base + u * 16, 16)
                acc_v[sla] = (acc_v[sla] + ring_v[sl]) * (1.0 / R)
            return 0

        lax.fori_loop(0, B2 // (16 * _UNROLL), body, 0)

    @pl.when(rhalf == 1)
    def _publish_even():
        pltpu.sync_copy(acc_v.at[pl.ds(0, B2)], shared.at[j])

    plsc.subcore_barrier()

    @pl.when(rhalf == 0)
    def _combine_even():
        _half_combine(0)

    plsc.subcore_barrier()

    @pl.when(rhalf == 1)
    def _publish_odd():
        pltpu.sync_copy(acc_v.at[pl.ds(B2, B2)], shared.at[j])

    plsc.subcore_barrier()

    @pl.when(rhalf == 0)
    def _combine_odd():
        _half_combine(B2)
        pltpu.sync_copy(acc_v, out.at[o])


def _sc_gather(sketch, hTi):
    mesh = plsc.VectorSubcoreMesh(
        core_axis_name="c", subcore_axis_name="s",
        num_cores=NUM_CORES, num_subcores=NUM_SUBCORES)
    f = pl.kernel(
        _sc_body,
        out_type=jax.ShapeDtypeStruct((OUT, B), jnp.float32),
        mesh=mesh,
        scratch_types=[
            pltpu.VMEM((_RING,), jnp.float32),
            pltpu.VMEM((B2,), jnp.int32),
            pltpu.VMEM((B,), jnp.float32),
            pltpu.VMEM_SHARED((OUT // 2, B2), jnp.float32),
            pltpu.SemaphoreType.DMA,
            pltpu.SemaphoreType.DMA,
            pltpu.SemaphoreType.DMA,
        ],
        compiler_params=pltpu.CompilerParams(needs_layout_passes=False),
    )
    return f(sketch, hTi)


def kernel(X, W, sketch):
    Wflat = W.reshape(R * K, D)
    M = jnp.asarray(_PACK)
    hTi = _compute_hashes(X, Wflat, M)
    out = _sc_gather(sketch, hTi)
    return out.T


# speedup vs baseline: 1.9720x; 1.0124x over previous
import jax
import jax.numpy as jnp
import numpy as np
from jax import lax
from jax.experimental import pallas as pl
from jax.experimental.pallas import tpu as pltpu
from jax.experimental.pallas import tpu_sc as plsc

K = 16
R = 16
D = 128
OUT = 16
NUM_CEL = 2 ** K
B = 16384
B2 = B // 2

NUM_CORES = 2
NUM_SUBCORES = 16

_PACK = np.zeros((R * K, R), dtype=np.float32)
for _r in range(R):
    for _k in range(K):
        _PACK[_r * K + _k, _r] = float(2 ** _k)

_BLK = 2048


def _hash_block(x_blk, w, m):
    projT = lax.dot_general(
        w, x_blk,
        dimension_numbers=(((1,), (1,)), ((), ())),
        preferred_element_type=jnp.float32)
    bits = (projT > 0).astype(jnp.float32)
    hT = lax.dot_general(
        m, bits,
        dimension_numbers=(((0,), (0,)), ((), ())),
        preferred_element_type=jnp.float32)
    return hT.astype(jnp.int32)


def _hash_body(xlo_ref, xhi_ref, w_ref, m_ref, out_ref):
    lo = _hash_block(xlo_ref[...], w_ref[...], m_ref[...])
    hi = _hash_block(xhi_ref[...], w_ref[...], m_ref[...])
    out_ref[...] = lo | lax.shift_left(hi, 16)


def _compute_hashes(X, Wflat, M):
    nblk = B2 // _BLK
    return pl.pallas_call(
        _hash_body,
        grid=(nblk,),
        in_specs=[
            pl.BlockSpec((_BLK, D), lambda i: (i, 0)),
            pl.BlockSpec((_BLK, D), lambda i, nblk=nblk: (nblk + i, 0)),
            pl.BlockSpec((R * K, D), lambda i: (0, 0)),
            pl.BlockSpec((R * K, R), lambda i: (0, 0)),
        ],
        out_specs=pl.BlockSpec((R, _BLK), lambda i: (0, i)),
        out_shape=jax.ShapeDtypeStruct((R, B2), jnp.int32),
    )(X, X, Wflat, M)


_UNROLL = 8
_NR = R // 2
_SUB = 32768
_SPR = NUM_CEL // _SUB
_NSLOT = 3
_RING = _NSLOT * _SUB


def _sc_body(sketch, hTi, out, ring_v, idx_v, acc_v, shared,
             r0_sem, r1_sem, i_sem):
    c = lax.axis_index("c")
    s = lax.axis_index("s")
    j = s // 2
    o = c * (OUT // 2) + j
    rhalf = s % 2
    r0 = rhalf * _NR

    rsem = [r0_sem, r1_sem]

    def idx_cp(t):
        return pltpu.make_async_copy(hTi.at[r0 + t], idx_v, i_sem)

    def sub_cp(n):
        t, m = divmod(n, _SPR)
        koff = (n % _NSLOT) * _SUB
        return pltpu.make_async_copy(
            sketch.at[o, r0 + t, pl.ds(m * _SUB, _SUB)],
            ring_v.at[pl.ds(koff, _SUB)],
            rsem[t % 2])

    @pl.when(rhalf == 1)
    def _stagger():
        pl.delay(2400)

    n_subs = _NR * _SPR
    idx_cp(0).start()
    for n in range(_NSLOT):
        sub_cp(n).start()
    fired = _NSLOT

    for t in range(_NR):
        idx_cp(t).wait()
        for m in range(_SPR):
            sub_cp(t * _SPR + m).wait()
        ib = idx_v
        first = (t == 0)
        base_off = ((_SPR * t) % _NSLOT) * _SUB

        @plsc.parallel_loop(0, B2, step=16, unroll=_UNROLL)
        def _gather_loop(i, ib=ib, first=first, base_off=base_off):
            sl = pl.ds(i, 16)
            w = ib[sl]
            idx_e = w & 0xFFFF
            idx_o = lax.shift_right_logical(w, 16)
            pe = base_off + idx_e
            pe = jnp.where(pe >= _RING, pe - _RING, pe)
            po = base_off + idx_o
            po = jnp.where(po >= _RING, po - _RING, po)
            ve = plsc.load_gather(ring_v, [pe])
            vo = plsc.load_gather(ring_v, [po])
            slo = pl.ds(B2 + i, 16)
            if first:
                acc_v[sl] = ve
                acc_v[slo] = vo
            else:
                plsc.addupdate(acc_v.at[sl], ve)
                plsc.addupdate(acc_v.at[slo], vo)

        nxt = min(fired + _SPR, n_subs)
        for n in range(fired, nxt):
            sub_cp(n).start()
        fired = nxt
        if t + 1 < _NR:
            idx_cp(t + 1).start()

    def _half_combine(acc_base):
        pltpu.sync_copy(shared.at[j], ring_v.at[pl.ds(0, B2)])

        def body(i, _):
            base = i * (16 * _UNROLL)
            for u in range(_UNROLL):
                sl = pl.ds(base + u * 16, 16)
                sla = pl.ds(acc_base + base + u * 16, 16)
                acc_v[sla] = (acc_v[sla] + ring_v[sl]) * (1.0 / R)
            return 0

        lax.fori_loop(0, B2 // (16 * _UNROLL), body, 0)

    @pl.when(rhalf == 1)
    def _publish_even():
        pltpu.sync_copy(acc_v.at[pl.ds(0, B2)], shared.at[j])

    plsc.subcore_barrier()

    @pl.when(rhalf == 0)
    def _combine_even():
        _half_combine(0)

    plsc.subcore_barrier()

    @pl.when(rhalf == 1)
    def _publish_odd():
        pltpu.sync_copy(acc_v.at[pl.ds(B2, B2)], shared.at[j])

    plsc.subcore_barrier()

    @pl.when(rhalf == 0)
    def _combine_odd():
        _half_combine(B2)
        pltpu.sync_copy(acc_v, out.at[o])


def _sc_gather(sketch, hTi):
    mesh = plsc.VectorSubcoreMesh(
        core_axis_name="c", subcore_axis_name="s",
        num_cores=NUM_CORES, num_subcores=NUM_SUBCORES)
    f = pl.kernel(
        _sc_body,
        out_type=jax.ShapeDtypeStruct((OUT, B), jnp.float32),
        mesh=mesh,
        scratch_types=[
            pltpu.VMEM((_RING,), jnp.float32),
            pltpu.VMEM((B2,), jnp.int32),
            pltpu.VMEM((B,), jnp.float32),
            pltpu.VMEM_SHARED((OUT // 2, B2), jnp.float32),
            pltpu.SemaphoreType.DMA,
            pltpu.SemaphoreType.DMA,
            pltpu.SemaphoreType.DMA,
        ],
        compiler_params=pltpu.CompilerParams(needs_layout_passes=False),
    )
    return f(sketch, hTi)


def kernel(X, W, sketch):
    Wflat = W.reshape(R * K, D)
    M = jnp.asarray(_PACK)
    hTi = _compute_hashes(X, Wflat, M)
    out = _sc_gather(sketch, hTi)
    return out.T
